# baseline (speedup 1.0000x reference)
import functools

import jax
import jax.numpy as jnp
from jax import lax
from jax.experimental import pallas as pl
from jax.experimental.pallas import tpu as pltpu
from jax.experimental.pallas import tpu_sc as plsc

D = 384
NC = 2
NS = 16
NW = NC * NS
CH = 128


@functools.lru_cache(maxsize=None)
def _build(B, n_out):
    bpw = B // NW
    nch = bpw // CH
    mesh = plsc.VectorSubcoreMesh(core_axis_name="c", subcore_axis_name="s")

    @functools.partial(
        pl.kernel,
        out_type=[jax.ShapeDtypeStruct((B, D), jnp.float32)] * n_out,
        mesh=mesh,
        scratch_types=[
            pltpu.VMEM((nch, CH), jnp.int32),
            pltpu.VMEM((2, CH, D), jnp.float32),
            pltpu.SemaphoreType.DMA,
            pltpu.SemaphoreType.DMA,
            pltpu.SemaphoreType.DMA,
            pltpu.SemaphoreType.DMA,
        ],
    )
    def emb1(idx_hbm, tbl, *rest):
        outs = rest[:n_out]
        idx_v, rows_v, g0, g1, w0, w1 = rest[n_out:]
        wid = lax.axis_index("s") * NC + lax.axis_index("c")
        base = wid * bpw
        pltpu.sync_copy(idx_hbm.at[pl.ds(wid * nch, nch)], idx_v)
        gsem = (g0, g1)
        wsem = (w0, w1)
        g = [None] * nch
        w = [None] * nch

        def fire_writes(j):
            slot = j % 2
            dst = pl.ds(base + j * CH, CH)
            return tuple(
                pltpu.async_copy(rows_v.at[slot], out.at[dst], wsem[slot])
                for out in outs
            )

        for j in range(nch):
            slot = j % 2
            if j >= 2:
                for d in w[j - 2]:
                    d.wait()
            g[j] = pltpu.async_copy(tbl.at[idx_v.at[j]], rows_v.at[slot], gsem[slot])
            if j >= 1:
                g[j - 1].wait()
                w[j - 1] = fire_writes(j - 1)
        g[nch - 1].wait()
        w[nch - 1] = fire_writes(nch - 1)
        for j in (nch - 2, nch - 1):
            if j >= 0 and w[j] is not None:
                for d in w[j]:
                    d.wait()

    return emb1


def kernel(inputs, emb0, emb1, emb2):
    batch, seq = inputs.shape
    B = batch * seq
    idx = inputs.reshape(B // CH, CH).astype(jnp.int32)
    v0 = _build(B, 1)(idx, emb0)[0]
    v1 = _build(B, 1)(idx, emb1)[0]
    v2, v2b = _build(B, 2)(idx, emb2)
    shp = (batch, seq, D)
    v0 = v0.reshape(shp)
    v1 = v1.reshape(shp)
    return (v0, v1, v2.reshape(shp), v0, v1, v2b.reshape(shp))


# speedup vs baseline: 1.1545x; 1.1545x over previous
import functools

import jax
import jax.numpy as jnp
from jax import lax
from jax.experimental import pallas as pl
from jax.experimental.pallas import tpu as pltpu
from jax.experimental.pallas import tpu_sc as plsc

D = 384
NC = 2
NS = 16
NW = NC * NS
CH = 128


@functools.lru_cache(maxsize=None)
def _build_gather(B, n_tables, dup):
    bpw = B // NW
    nch = bpw // CH
    n_out = n_tables * dup
    mesh = plsc.VectorSubcoreMesh(core_axis_name="c", subcore_axis_name="s")

    @functools.partial(
        pl.kernel,
        out_type=[jax.ShapeDtypeStruct((B, D), jnp.float32)] * n_out,
        mesh=mesh,
        scratch_types=[
            pltpu.VMEM((nch, CH), jnp.int32),
            pltpu.VMEM((2, CH, D), jnp.float32),
            pltpu.SemaphoreType.DMA,
            pltpu.SemaphoreType.DMA,
            pltpu.SemaphoreType.DMA,
            pltpu.SemaphoreType.DMA,
        ],
    )
    def emb(idx_hbm, *rest):
        tables = rest[:n_tables]
        outs = rest[n_tables:n_tables + n_out]
        idx_v, rows_v, g0, g1, w0, w1 = rest[n_tables + n_out:]
        wid = lax.axis_index("s") * NC + lax.axis_index("c")
        base = wid * bpw
        pltpu.sync_copy(idx_hbm.at[pl.ds(wid * nch, nch)], idx_v)
        gsem = (g0, g1)
        wsem = (w0, w1)
        units = [(t, j) for t in range(n_tables) for j in range(nch)]
        n = len(units)
        g = [None] * n
        w = [None] * n

        def fire_writes(u):
            t, j = units[u]
            slot = u % 2
            dst = pl.ds(base + j * CH, CH)
            return tuple(
                pltpu.async_copy(rows_v.at[slot], outs[t + k * n_tables].at[dst],
                                 wsem[slot])
                for k in range(dup)
            )

        for u, (t, j) in enumerate(units):
            slot = u % 2
            if u >= 2:
                for d in w[u - 2]:
                    d.wait()
            g[u] = pltpu.async_copy(tables[t].at[idx_v.at[j]], rows_v.at[slot],
                                    gsem[slot])
            if u >= 1:
                g[u - 1].wait()
                w[u - 1] = fire_writes(u - 1)
        g[n - 1].wait()
        w[n - 1] = fire_writes(n - 1)
        for u in (n - 2, n - 1):
            for d in w[u]:
                d.wait()

    return emb


@functools.lru_cache(maxsize=None)
def _build_tc_copy(B):
    rows = 512
    grid = (B // rows,)
    spec = pl.BlockSpec((rows, D), lambda i: (i, 0))

    def body(a, b, oa, ob):
        oa[...] = a[...]
        ob[...] = b[...]

    return pl.pallas_call(
        body,
        grid=grid,
        in_specs=[spec, spec],
        out_specs=[spec, spec],
        out_shape=[jax.ShapeDtypeStruct((B, D), jnp.float32)] * 2,
    )


def kernel(inputs, emb0, emb1, emb2):
    batch, seq = inputs.shape
    B = batch * seq
    idx = inputs.reshape(B // CH, CH).astype(jnp.int32)
    v0, v1 = _build_gather(B, 2, 1)(idx, emb0, emb1)
    v3, v4 = _build_tc_copy(B)(v0, v1)
    v2, v5 = _build_gather(B, 1, 2)(idx, emb2)
    shp = (batch, seq, D)
    return tuple(v.reshape(shp) for v in (v0, v1, v2, v3, v4, v5))


# speedup vs baseline: 1.2776x; 1.1066x over previous
import functools

import jax
import jax.numpy as jnp
from jax import lax
from jax.experimental import pallas as pl
from jax.experimental.pallas import tpu as pltpu
from jax.experimental.pallas import tpu_sc as plsc

D = 384
NC = 2
NS = 16
NW = NC * NS
CH = 64


@functools.lru_cache(maxsize=None)
def _build(B):
    bpw = B // NW
    nch = bpw // CH
    mesh = plsc.VectorSubcoreMesh(core_axis_name="c", subcore_axis_name="s")

    @functools.partial(
        pl.kernel,
        out_type=[jax.ShapeDtypeStruct((B, D), jnp.float32)] * 6,
        mesh=mesh,
        scratch_types=[
            pltpu.VMEM((nch, CH), jnp.int32),
            pltpu.VMEM((2, CH, D), jnp.float32),
            pltpu.SemaphoreType.DMA,
            pltpu.SemaphoreType.DMA,
            pltpu.SemaphoreType.DMA,
            pltpu.SemaphoreType.DMA,
        ],
    )
    def emb3(idx_hbm, t0, t1, t2, o0, o1, o2, o3, o4, o5, idx_v, rows_v,
             g0, g1, w0, w1):
        wid = lax.axis_index("s") * NC + lax.axis_index("c")
        base = wid * bpw
        pltpu.sync_copy(idx_hbm.at[pl.ds(wid * nch, nch)], idx_v)
        gsem = (g0, g1)
        wsem = (w0, w1)
        units = [
            (tbl, outa, outb, j)
            for tbl, outa, outb in ((t0, o0, o3), (t1, o1, o4), (t2, o2, o5))
            for j in range(nch)
        ]
        n = len(units)
        g = [None] * n
        w = [None] * n

        def fire_writes(u):
            tbl, outa, outb, j = units[u]
            slot = u % 2
            dst = pl.ds(base + j * CH, CH)
            return (
                pltpu.async_copy(rows_v.at[slot], outa.at[dst], wsem[slot]),
                pltpu.async_copy(rows_v.at[slot], outb.at[dst], wsem[slot]),
            )

        for u, (tbl, outa, outb, j) in enumerate(units):
            slot = u % 2
            if u >= 2:
                w[u - 2][0].wait()
                w[u - 2][1].wait()
            g[u] = pltpu.async_copy(tbl.at[idx_v.at[j]], rows_v.at[slot], gsem[slot])
            if u >= 1:
                g[u - 1].wait()
                w[u - 1] = fire_writes(u - 1)
        g[n - 1].wait()
        w[n - 1] = fire_writes(n - 1)
        for d in w[n - 2]:
            d.wait()
        for d in w[n - 1]:
            d.wait()

    return emb3


def kernel(inputs, emb0, emb1, emb2):
    batch, seq = inputs.shape
    B = batch * seq
    idx = inputs.reshape(B // CH, CH).astype(jnp.int32)
    outs = _build(B)(idx, emb0, emb1, emb2)
    return tuple(o.reshape(batch, seq, D) for o in outs)


# speedup vs baseline: 1.3315x; 1.0422x over previous
import functools

import jax
import jax.numpy as jnp
from jax import lax
from jax.experimental import pallas as pl
from jax.experimental.pallas import tpu as pltpu
from jax.experimental.pallas import tpu_sc as plsc

D = 384
NC = 2
NS = 16
NW = NC * NS
CH = 128


@functools.lru_cache(maxsize=None)
def _build(batch, seq):
    B = batch * seq
    bpw = B // NW
    nch = bpw // CH
    wpr = seq // bpw
    mesh = plsc.VectorSubcoreMesh(core_axis_name="c", subcore_axis_name="s")

    @functools.partial(
        pl.kernel,
        out_type=[jax.ShapeDtypeStruct((B, D), jnp.float32)] * 6,
        mesh=mesh,
        scratch_types=[
            pltpu.VMEM((nch, CH), jnp.int32),
            pltpu.VMEM((2, CH, D), jnp.float32),
            pltpu.SemaphoreType.DMA,
            pltpu.SemaphoreType.DMA,
            pltpu.SemaphoreType.DMA,
            pltpu.SemaphoreType.DMA,
        ],
    )
    def emb3(idx_hbm, t0, t1, t2, o0, o1, o2, o3, o4, o5, idx_v, rows_v,
             g0, g1, w0, w1):
        wid = lax.axis_index("s") * NC + lax.axis_index("c")
        base = wid * bpw
        row = wid // wpr
        col = (wid % wpr) * bpw
        for j in range(nch):
            pltpu.sync_copy(idx_hbm.at[row, pl.ds(col + j * CH, CH)], idx_v.at[j])
        gsem = (g0, g1)
        wsem = (w0, w1)
        units = [
            (tbl, outa, outb, j)
            for tbl, outa, outb in ((t0, o0, o3), (t1, o1, o4), (t2, o2, o5))
            for j in range(nch)
        ]
        n = len(units)
        g = [None] * n
        w = [None] * n

        def fire_writes(u):
            tbl, outa, outb, j = units[u]
            slot = u % 2
            dst = pl.ds(base + j * CH, CH)
            return (
                pltpu.async_copy(rows_v.at[slot], outa.at[dst], wsem[slot]),
                pltpu.async_copy(rows_v.at[slot], outb.at[dst], wsem[slot]),
            )

        for u, (tbl, outa, outb, j) in enumerate(units):
            slot = u % 2
            if u >= 2:
                w[u - 2][0].wait()
                w[u - 2][1].wait()
            g[u] = pltpu.async_copy(tbl.at[idx_v.at[j]], rows_v.at[slot], gsem[slot])
            if u >= 1:
                g[u - 1].wait()
                w[u - 1] = fire_writes(u - 1)
        g[n - 1].wait()
        w[n - 1] = fire_writes(n - 1)
        for d in w[n - 2]:
            d.wait()
        for d in w[n - 1]:
            d.wait()

    return emb3


def kernel(inputs, emb0, emb1, emb2):
    batch, seq = inputs.shape
    outs = _build(batch, seq)(inputs.astype(jnp.int32), emb0, emb1, emb2)
    return tuple(o.reshape(batch, seq, D) for o in outs)
